# baseline (speedup 1.0000x reference)
import functools

import jax
import jax.numpy as jnp
from jax import lax
from jax.experimental import pallas as pl
from jax.experimental.pallas import tpu as pltpu
from jax.experimental.pallas import tpu_sc as plsc


def _detile_body(x_ref, o_ref):
    y = x_ref[...]
    o_ref[:, 0:64] = y[:, 0:256].T
    o_ref[:, 64:128] = y[:, 256:512].T


def _make_detile(v: int, d: int):
    assert d == 64
    bc = 512
    n_blocks = -(-v // bc)
    out_rows = n_blocks * 256

    return pl.pallas_call(
        _detile_body,
        grid=(n_blocks,),
        in_specs=[pl.BlockSpec((d, bc), lambda i: (0, i))],
        out_specs=pl.BlockSpec((256, 128), lambda i: (i, 0)),
        out_shape=jax.ShapeDtypeStruct((out_rows, 128), jnp.float32),
    )


def _make_gather(batch: int, seq: int, d: int):
    info = plsc.get_sparse_core_info()
    nc, ns = info.num_cores, info.num_subcores
    nw = nc * ns
    n_rows = batch * seq
    assert n_rows % nw == 0
    b_per_w = n_rows // nw
    ch = seq
    nbuf = 4
    la = 2
    assert b_per_w % (ch * nbuf) == 0
    n_chunks = b_per_w // ch
    n_rounds = n_chunks // nbuf

    @functools.partial(
        pl.kernel,
        mesh=plsc.VectorSubcoreMesh(core_axis_name="c", subcore_axis_name="s"),
        out_type=jax.ShapeDtypeStruct((batch, seq, d), jnp.float32),
        compiler_params=pltpu.CompilerParams(use_tc_tiling_on_sc=False),
        scratch_types=[
            pltpu.VMEM((b_per_w,), jnp.int32),
            *[pltpu.VMEM((ch, d), jnp.float32) for _ in range(nbuf)],
            pltpu.SemaphoreType.DMA((nbuf,)),
            pltpu.SemaphoreType.DMA((nbuf,)),
        ],
    )
    def gather(table_hbm, idx_hbm, out3_hbm, idx_v, r0, r1, r2, r3, sg, sw):
        rows = (r0, r1, r2, r3)
        wid = lax.axis_index("s") * nc + lax.axis_index("c")
        base = wid * b_per_w
        brow0 = wid * n_chunks
        pltpu.sync_copy(idx_hbm.at[pl.ds(base, b_per_w)], idx_v)

        def start_gather(c, b):
            pltpu.make_async_copy(
                table_hbm.at[idx_v.at[pl.ds(c * ch, ch)]], rows[b], sg.at[b]
            ).start()

        def wait_gather(b):
            pltpu.make_async_copy(
                table_hbm.at[idx_v.at[pl.ds(0, ch)]], rows[b], sg.at[b]
            ).wait()

        def start_wb(c, b):
            pltpu.make_async_copy(rows[b], out3_hbm.at[brow0 + c], sw.at[b]).start()

        def wait_wb(b):
            pltpu.make_async_copy(rows[b], out3_hbm.at[brow0], sw.at[b]).wait()

        for b in range(la):
            start_gather(b, b)

        def round_body(r, carry):
            for b in range(nbuf):
                c = r * nbuf + b
                b2 = (b + la) % nbuf
                wait_gather(b)
                start_wb(c, b)

                @pl.when(c + la < n_chunks)
                def _():
                    @pl.when(c >= nbuf - la)
                    def _():
                        wait_wb(b2)

                    start_gather(c + la, b2)

            return carry

        lax.fori_loop(0, n_rounds, round_body, 0)

        for b in range(nbuf - la):
            wait_wb((n_chunks - (nbuf - la) + b) % nbuf)

    return gather


def kernel(features, word_embeddings_weight):
    b, s = features.shape
    v, d = word_embeddings_weight.shape
    idx = features.reshape(b * s).astype(jnp.int32)
    idx2 = 2 * (256 * (idx // 512) + idx % 256) + (idx % 512) // 256
    tpaired = _make_detile(v, d)(word_embeddings_weight.T)
    table_lin = tpaired.reshape(2 * tpaired.shape[0], d)
    gather = _make_gather(b, s, d)
    return gather(table_lin, idx2)


# speedup vs baseline: 2.0276x; 2.0276x over previous
import functools

import jax
import jax.numpy as jnp
from jax import lax
from jax.experimental import pallas as pl
from jax.experimental.pallas import tpu as pltpu
from jax.experimental.pallas import tpu_sc as plsc


_DETILE_GROUPS = 16


def _detile_body(x_ref, o_ref):
    y = x_ref[...]
    for k in range(_DETILE_GROUPS):
        o_ref[256 * k : 256 * (k + 1), 0:64] = y[:, 512 * k : 512 * k + 256].T
        o_ref[256 * k : 256 * (k + 1), 64:128] = y[:, 512 * k + 256 : 512 * (k + 1)].T


def _make_detile(v: int, d: int):
    assert d == 64
    bc = 512 * _DETILE_GROUPS
    n_blocks = -(-v // bc)
    out_rows = n_blocks * (bc // 2)

    return pl.pallas_call(
        _detile_body,
        grid=(n_blocks,),
        in_specs=[pl.BlockSpec((d, bc), lambda i: (0, i))],
        out_specs=pl.BlockSpec((bc // 2, 128), lambda i: (i, 0)),
        out_shape=jax.ShapeDtypeStruct((out_rows, 128), jnp.float32),
    )


def _make_gather(batch: int, seq: int, d: int):
    info = plsc.get_sparse_core_info()
    nc, ns = info.num_cores, info.num_subcores
    nw = nc * ns
    n_rows = batch * seq
    assert n_rows % nw == 0
    b_per_w = n_rows // nw
    ch = seq
    nbuf = 4
    la = 2
    assert b_per_w % (ch * nbuf) == 0
    n_chunks = b_per_w // ch
    n_rounds = n_chunks // nbuf

    @functools.partial(
        pl.kernel,
        mesh=plsc.VectorSubcoreMesh(core_axis_name="c", subcore_axis_name="s"),
        out_type=jax.ShapeDtypeStruct((batch, seq, d), jnp.float32),
        compiler_params=pltpu.CompilerParams(use_tc_tiling_on_sc=False),
        scratch_types=[
            pltpu.VMEM((b_per_w,), jnp.int32),
            *[pltpu.VMEM((ch, d), jnp.float32) for _ in range(nbuf)],
            pltpu.SemaphoreType.DMA((nbuf,)),
            pltpu.SemaphoreType.DMA((nbuf,)),
        ],
    )
    def gather(table_hbm, idx_hbm, out3_hbm, idx_v, r0, r1, r2, r3, sg, sw):
        rows = (r0, r1, r2, r3)
        wid = lax.axis_index("s") * nc + lax.axis_index("c")
        base = wid * b_per_w
        brow0 = wid * n_chunks
        pltpu.sync_copy(idx_hbm.at[pl.ds(base, b_per_w)], idx_v)

        def start_gather(c, b):
            pltpu.make_async_copy(
                table_hbm.at[idx_v.at[pl.ds(c * ch, ch)]], rows[b], sg.at[b]
            ).start()

        def wait_gather(b):
            pltpu.make_async_copy(
                table_hbm.at[idx_v.at[pl.ds(0, ch)]], rows[b], sg.at[b]
            ).wait()

        def start_wb(c, b):
            pltpu.make_async_copy(rows[b], out3_hbm.at[brow0 + c], sw.at[b]).start()

        def wait_wb(b):
            pltpu.make_async_copy(rows[b], out3_hbm.at[brow0], sw.at[b]).wait()

        for b in range(la):
            start_gather(b, b)

        def round_body(r, carry):
            for b in range(nbuf):
                c = r * nbuf + b
                b2 = (b + la) % nbuf
                wait_gather(b)
                start_wb(c, b)

                @pl.when(c + la < n_chunks)
                def _():
                    @pl.when(c >= nbuf - la)
                    def _():
                        wait_wb(b2)

                    start_gather(c + la, b2)

            return carry

        lax.fori_loop(0, n_rounds, round_body, 0)

        for b in range(nbuf - la):
            wait_wb((n_chunks - (nbuf - la) + b) % nbuf)

    return gather


def kernel(features, word_embeddings_weight):
    b, s = features.shape
    v, d = word_embeddings_weight.shape
    idx = features.reshape(b * s).astype(jnp.int32)
    idx2 = 2 * (256 * (idx // 512) + idx % 256) + (idx % 512) // 256
    tpaired = _make_detile(v, d)(word_embeddings_weight.T)
    table_lin = tpaired.reshape(2 * tpaired.shape[0], d)
    gather = _make_gather(b, s, d)
    return gather(table_lin, idx2)


# speedup vs baseline: 2.1018x; 1.0366x over previous
import functools

import jax
import jax.numpy as jnp
from jax import lax
from jax.experimental import pallas as pl
from jax.experimental.pallas import tpu as pltpu
from jax.experimental.pallas import tpu_sc as plsc


_DETILE_GROUPS = 32


def _detile_body(x_ref, o_ref):
    y = x_ref[...]
    for k in range(_DETILE_GROUPS):
        o_ref[256 * k : 256 * (k + 1), 0:64] = y[:, 512 * k : 512 * k + 256].T
        o_ref[256 * k : 256 * (k + 1), 64:128] = y[:, 512 * k + 256 : 512 * (k + 1)].T


def _make_detile(v: int, d: int):
    assert d == 64
    bc = 512 * _DETILE_GROUPS
    n_blocks = -(-v // bc)
    out_rows = n_blocks * (bc // 2)

    return pl.pallas_call(
        _detile_body,
        grid=(n_blocks,),
        in_specs=[pl.BlockSpec((d, bc), lambda i: (0, i))],
        out_specs=pl.BlockSpec((bc // 2, 128), lambda i: (i, 0)),
        out_shape=jax.ShapeDtypeStruct((out_rows, 128), jnp.float32),
    )


def _make_gather(batch: int, seq: int, d: int):
    info = plsc.get_sparse_core_info()
    nc, ns = info.num_cores, info.num_subcores
    nw = nc * ns
    n_rows = batch * seq
    assert n_rows % nw == 0
    b_per_w = n_rows // nw
    ch = seq
    nbuf = 4
    la = 2
    assert b_per_w % (ch * nbuf) == 0
    n_chunks = b_per_w // ch
    n_rounds = n_chunks // nbuf

    @functools.partial(
        pl.kernel,
        mesh=plsc.VectorSubcoreMesh(core_axis_name="c", subcore_axis_name="s"),
        out_type=jax.ShapeDtypeStruct((batch, seq, d), jnp.float32),
        compiler_params=pltpu.CompilerParams(use_tc_tiling_on_sc=False),
        scratch_types=[
            pltpu.VMEM((b_per_w,), jnp.int32),
            *[pltpu.VMEM((ch, d), jnp.float32) for _ in range(nbuf)],
            pltpu.SemaphoreType.DMA((nbuf,)),
            pltpu.SemaphoreType.DMA((nbuf,)),
        ],
    )
    def gather(table_hbm, idx_hbm, out3_hbm, idx_v, r0, r1, r2, r3, sg, sw):
        rows = (r0, r1, r2, r3)
        wid = lax.axis_index("s") * nc + lax.axis_index("c")
        base = wid * b_per_w
        brow0 = wid * n_chunks
        pltpu.sync_copy(idx_hbm.at[pl.ds(base, b_per_w)], idx_v)

        def start_gather(c, b):
            pltpu.make_async_copy(
                table_hbm.at[idx_v.at[pl.ds(c * ch, ch)]], rows[b], sg.at[b]
            ).start()

        def wait_gather(b):
            pltpu.make_async_copy(
                table_hbm.at[idx_v.at[pl.ds(0, ch)]], rows[b], sg.at[b]
            ).wait()

        def start_wb(c, b):
            pltpu.make_async_copy(rows[b], out3_hbm.at[brow0 + c], sw.at[b]).start()

        def wait_wb(b):
            pltpu.make_async_copy(rows[b], out3_hbm.at[brow0], sw.at[b]).wait()

        for b in range(la):
            start_gather(b, b)

        def round_body(r, carry):
            for b in range(nbuf):
                c = r * nbuf + b
                b2 = (b + la) % nbuf
                wait_gather(b)
                start_wb(c, b)

                @pl.when(c + la < n_chunks)
                def _():
                    @pl.when(c >= nbuf - la)
                    def _():
                        wait_wb(b2)

                    start_gather(c + la, b2)

            return carry

        lax.fori_loop(0, n_rounds, round_body, 0)

        for b in range(nbuf - la):
            wait_wb((n_chunks - (nbuf - la) + b) % nbuf)

    return gather


def kernel(features, word_embeddings_weight):
    b, s = features.shape
    v, d = word_embeddings_weight.shape
    idx = features.reshape(b * s).astype(jnp.int32)
    idx2 = 2 * (256 * (idx // 512) + idx % 256) + (idx % 512) // 256
    tpaired = _make_detile(v, d)(word_embeddings_weight.T)
    table_lin = tpaired.reshape(2 * tpaired.shape[0], d)
    gather = _make_gather(b, s, d)
    return gather(table_lin, idx2)


# speedup vs baseline: 2.1349x; 1.0158x over previous
import functools

import jax
import jax.numpy as jnp
from jax import lax
from jax.experimental import pallas as pl
from jax.experimental.pallas import tpu as pltpu
from jax.experimental.pallas import tpu_sc as plsc


_DETILE_GROUPS = 64


def _detile_body(x_ref, o_ref):
    y = x_ref[...]
    for k in range(_DETILE_GROUPS):
        o_ref[256 * k : 256 * (k + 1), 0:64] = y[:, 512 * k : 512 * k + 256].T
        o_ref[256 * k : 256 * (k + 1), 64:128] = y[:, 512 * k + 256 : 512 * (k + 1)].T


def _make_detile(v: int, d: int):
    assert d == 64
    bc = 512 * _DETILE_GROUPS
    n_blocks = -(-v // bc)
    out_rows = n_blocks * (bc // 2)

    return pl.pallas_call(
        _detile_body,
        grid=(n_blocks,),
        in_specs=[pl.BlockSpec((d, bc), lambda i: (0, i))],
        out_specs=pl.BlockSpec((bc // 2, 128), lambda i: (i, 0)),
        out_shape=jax.ShapeDtypeStruct((out_rows, 128), jnp.float32),
    )


def _make_gather(batch: int, seq: int, d: int):
    info = plsc.get_sparse_core_info()
    nc, ns = info.num_cores, info.num_subcores
    nw = nc * ns
    n_rows = batch * seq
    assert n_rows % nw == 0
    b_per_w = n_rows // nw
    ch = seq
    nbuf = 4
    la = 2
    assert b_per_w % (ch * nbuf) == 0
    n_chunks = b_per_w // ch
    n_rounds = n_chunks // nbuf

    @functools.partial(
        pl.kernel,
        mesh=plsc.VectorSubcoreMesh(core_axis_name="c", subcore_axis_name="s"),
        out_type=jax.ShapeDtypeStruct((batch, seq, d), jnp.float32),
        compiler_params=pltpu.CompilerParams(use_tc_tiling_on_sc=False),
        scratch_types=[
            pltpu.VMEM((b_per_w,), jnp.int32),
            *[pltpu.VMEM((ch, d), jnp.float32) for _ in range(nbuf)],
            pltpu.SemaphoreType.DMA((nbuf,)),
            pltpu.SemaphoreType.DMA((nbuf,)),
        ],
    )
    def gather(table_hbm, idx_hbm, out3_hbm, idx_v, r0, r1, r2, r3, sg, sw):
        rows = (r0, r1, r2, r3)
        wid = lax.axis_index("s") * nc + lax.axis_index("c")
        base = wid * b_per_w
        brow0 = wid * n_chunks
        pltpu.sync_copy(idx_hbm.at[pl.ds(base, b_per_w)], idx_v)

        def start_gather(c, b):
            pltpu.make_async_copy(
                table_hbm.at[idx_v.at[pl.ds(c * ch, ch)]], rows[b], sg.at[b]
            ).start()

        def wait_gather(b):
            pltpu.make_async_copy(
                table_hbm.at[idx_v.at[pl.ds(0, ch)]], rows[b], sg.at[b]
            ).wait()

        def start_wb(c, b):
            pltpu.make_async_copy(rows[b], out3_hbm.at[brow0 + c], sw.at[b]).start()

        def wait_wb(b):
            pltpu.make_async_copy(rows[b], out3_hbm.at[brow0], sw.at[b]).wait()

        for b in range(la):
            start_gather(b, b)

        def round_body(r, carry):
            for b in range(nbuf):
                c = r * nbuf + b
                b2 = (b + la) % nbuf
                wait_gather(b)
                start_wb(c, b)

                @pl.when(c + la < n_chunks)
                def _():
                    @pl.when(c >= nbuf - la)
                    def _():
                        wait_wb(b2)

                    start_gather(c + la, b2)

            return carry

        lax.fori_loop(0, n_rounds, round_body, 0)

        for b in range(nbuf - la):
            wait_wb((n_chunks - (nbuf - la) + b) % nbuf)

    return gather


def kernel(features, word_embeddings_weight):
    b, s = features.shape
    v, d = word_embeddings_weight.shape
    idx = features.reshape(b * s).astype(jnp.int32)
    idx2 = 2 * (256 * (idx // 512) + idx % 256) + (idx % 512) // 256
    tpaired = _make_detile(v, d)(word_embeddings_weight.T)
    table_lin = tpaired.reshape(2 * tpaired.shape[0], d)
    gather = _make_gather(b, s, d)
    return gather(table_lin, idx2)


# speedup vs baseline: 2.1448x; 1.0046x over previous
import functools

import jax
import jax.numpy as jnp
from jax import lax
from jax.experimental import pallas as pl
from jax.experimental.pallas import tpu as pltpu
from jax.experimental.pallas import tpu_sc as plsc


_DETILE_GROUPS = 64


def _detile_body(x_ref, o_ref):
    y = x_ref[...]
    for k in range(_DETILE_GROUPS):
        o_ref[256 * k : 256 * (k + 1), 0:64] = y[:, 512 * k : 512 * k + 256].T
        o_ref[256 * k : 256 * (k + 1), 64:128] = y[:, 512 * k + 256 : 512 * (k + 1)].T


def _make_detile(v: int, d: int):
    assert d == 64
    bc = 512 * _DETILE_GROUPS
    n_blocks = -(-v // bc)
    out_rows = n_blocks * (bc // 2)

    return pl.pallas_call(
        _detile_body,
        grid=(n_blocks,),
        in_specs=[pl.BlockSpec((d, bc), lambda i: (0, i))],
        out_specs=pl.BlockSpec((bc // 2, 128), lambda i: (i, 0)),
        out_shape=jax.ShapeDtypeStruct((out_rows, 128), jnp.float32),
    )


def _make_gather(batch: int, seq: int, d: int):
    info = plsc.get_sparse_core_info()
    nc, ns = info.num_cores, info.num_subcores
    nw = nc * ns
    n_rows = batch * seq
    assert n_rows % nw == 0
    b_per_w = n_rows // nw
    ch = seq
    nbuf = 8
    la = 4
    assert b_per_w % (ch * nbuf) == 0
    n_chunks = b_per_w // ch
    n_rounds = n_chunks // nbuf

    @functools.partial(
        pl.kernel,
        mesh=plsc.VectorSubcoreMesh(core_axis_name="c", subcore_axis_name="s"),
        out_type=jax.ShapeDtypeStruct((batch, seq, d), jnp.float32),
        compiler_params=pltpu.CompilerParams(use_tc_tiling_on_sc=False),
        scratch_types=[
            pltpu.VMEM((b_per_w,), jnp.int32),
            *[pltpu.VMEM((ch, d), jnp.float32) for _ in range(nbuf)],
            pltpu.SemaphoreType.DMA((nbuf,)),
            pltpu.SemaphoreType.DMA((nbuf,)),
        ],
    )
    def gather(table_hbm, idx_hbm, out3_hbm, idx_v, r0, r1, r2, r3, r4, r5, r6, r7, sg, sw):
        rows = (r0, r1, r2, r3, r4, r5, r6, r7)
        wid = lax.axis_index("s") * nc + lax.axis_index("c")
        base = wid * b_per_w
        brow0 = wid * n_chunks
        pltpu.sync_copy(idx_hbm.at[pl.ds(base, b_per_w)], idx_v)

        def start_gather(c, b):
            pltpu.make_async_copy(
                table_hbm.at[idx_v.at[pl.ds(c * ch, ch)]], rows[b], sg.at[b]
            ).start()

        def wait_gather(b):
            pltpu.make_async_copy(
                table_hbm.at[idx_v.at[pl.ds(0, ch)]], rows[b], sg.at[b]
            ).wait()

        def start_wb(c, b):
            pltpu.make_async_copy(rows[b], out3_hbm.at[brow0 + c], sw.at[b]).start()

        def wait_wb(b):
            pltpu.make_async_copy(rows[b], out3_hbm.at[brow0], sw.at[b]).wait()

        for b in range(la):
            start_gather(b, b)

        def round_body(r, carry):
            for b in range(nbuf):
                c = r * nbuf + b
                b2 = (b + la) % nbuf
                wait_gather(b)
                start_wb(c, b)

                @pl.when(c + la < n_chunks)
                def _():
                    @pl.when(c >= nbuf - la)
                    def _():
                        wait_wb(b2)

                    start_gather(c + la, b2)

            return carry

        lax.fori_loop(0, n_rounds, round_body, 0)

        for b in range(nbuf - la):
            wait_wb((n_chunks - (nbuf - la) + b) % nbuf)

    return gather


def kernel(features, word_embeddings_weight):
    b, s = features.shape
    v, d = word_embeddings_weight.shape
    idx = features.reshape(b * s).astype(jnp.int32)
    idx2 = 2 * (256 * (idx // 512) + idx % 256) + (idx % 512) // 256
    tpaired = _make_detile(v, d)(word_embeddings_weight.T)
    table_lin = tpaired.reshape(2 * tpaired.shape[0], d)
    gather = _make_gather(b, s, d)
    return gather(table_lin, idx2)
